# baseline (speedup 1.0000x reference)
import functools

import jax
import jax.numpy as jnp
from jax import lax
from jax.experimental import pallas as pl
from jax.experimental.pallas import tpu as pltpu
from jax.experimental.pallas import tpu_sc as plsc

_ORDER = 50
_EMBED = 64
_PAD = 64
_L = 16


def _body(idx_hbm, table_hbm, pos_hbm, out_hbm, idx_v, pos_v, w_v, rows_v,
          out_v, sem):
    c = lax.axis_index("c")
    s = lax.axis_index("s")

    @pl.when(jnp.logical_and(c == 0, s == 0))
    def _():
        zero_i = jnp.zeros((_L,), jnp.int32)
        zero_f = jnp.zeros((_L,), jnp.float32)
        idx_v[pl.ds(_PAD - _L, _L)] = zero_i
        pos_v[pl.ds(_PAD - _L, _L)] = zero_f
        pltpu.sync_copy(idx_hbm, idx_v.at[pl.ds(0, _ORDER)])
        pltpu.sync_copy(pos_hbm, pos_v.at[pl.ds(0, _ORDER)])

        for ci in range(_PAD // _L):
            sl = pl.ds(ci * _L, _L)
            iv = idx_v[sl]
            w_v[sl] = jnp.where(iv != -1, pos_v[sl], jnp.zeros((_L,),
                                                               jnp.float32))
            idx_v[sl] = jnp.maximum(iv, 0)

        pltpu.async_copy(table_hbm.at[idx_v], rows_v, sem).wait()

        acc = [jnp.zeros((_L,), jnp.float32) for _ in range(_EMBED // _L)]
        for ci in range(_PAD // _L):
            w_chunk = w_v[pl.ds(ci * _L, _L)]
            for j in range(_L):
                i = ci * _L + j
                wi = w_chunk[j]
                for cc in range(_EMBED // _L):
                    acc[cc] = acc[cc] + wi * rows_v[i, pl.ds(cc * _L, _L)]

        wsum = jnp.zeros((_L,), jnp.float32)
        for ci in range(_PAD // _L):
            wsum = wsum + w_v[pl.ds(ci * _L, _L)]
        total = wsum[0]
        for j in range(1, _L):
            total = total + wsum[j]
        inv = jnp.ones((_L,), jnp.float32) / jnp.full((_L,), total,
                                                      jnp.float32)
        for cc in range(_EMBED // _L):
            out_v[pl.ds(cc * _L, _L)] = acc[cc] * inv
        pltpu.sync_copy(out_v, out_hbm)


@jax.jit
def kernel(partial_path_candidate, objects_embeds, positional_encoding):
    mesh = plsc.VectorSubcoreMesh(core_axis_name="c", subcore_axis_name="s")
    k = functools.partial(
        pl.kernel,
        out_type=jax.ShapeDtypeStruct((_EMBED,), jnp.float32),
        mesh=mesh,
        scratch_types=[
            pltpu.VMEM((_PAD,), jnp.int32),
            pltpu.VMEM((_PAD,), jnp.float32),
            pltpu.VMEM((_PAD,), jnp.float32),
            pltpu.VMEM((_PAD, _EMBED), jnp.float32),
            pltpu.VMEM((_EMBED,), jnp.float32),
            pltpu.SemaphoreType.DMA,
        ],
        compiler_params=pltpu.CompilerParams(use_tc_tiling_on_sc=False),
    )(_body)
    return k(partial_path_candidate, objects_embeds, positional_encoding)
